# baseline (speedup 1.0000x reference)
import functools

import jax
import jax.numpy as jnp
from jax import lax
from jax.experimental import pallas as pl
from jax.experimental.pallas import tpu as pltpu
from jax.experimental.pallas import tpu_sc as plsc

B = 16384
D = 64
NC = 2
NS = 16
L = 16
NW = NC * NS
BPW = B // NW
NG = BPW // L
CH = 128
NCH = BPW // CH
GPC = CH // L


def _sc_body(user_hbm, item_hbm, ut_hbm, it_hbm, out_hbm,
             uidx_v, iidx_v, tu_v, ti_v, ubuf_v, ibuf_v, out_v, scr_v,
             sem_u, sem_i):
    wid = lax.axis_index("s") * NC + lax.axis_index("c")
    base = wid * BPW

    pltpu.sync_copy(user_hbm.at[pl.ds(base, BPW)], uidx_v)
    pltpu.sync_copy(item_hbm.at[pl.ds(base, BPW)], iidx_v)

    def tix_body(k, _):
        tu_v[pl.ds(k * L, L)] = uidx_v[pl.ds(k * L, L)] // 2
        ti_v[pl.ds(k * L, L)] = iidx_v[pl.ds(k * L, L)] // 2
        return 0

    lax.fori_loop(0, NG, tix_body, 0)

    lanes = lax.iota(jnp.int32, L)

    for c in range(NCH):
        cu = pltpu.async_copy(
            ut_hbm.at[tu_v.at[pl.ds(c * CH, CH)]], ubuf_v, sem_u)
        ci = pltpu.async_copy(
            it_hbm.at[ti_v.at[pl.ds(c * CH, CH)]], ibuf_v, sem_i)
        cu.wait()
        ci.wait()

        def group_body(g, _, c=c):
            rbase = c * CH + g * L
            ivu = uidx_v[pl.ds(rbase, L)]
            ivi = iidx_v[pl.ds(rbase, L)]
            for r16 in range(L):
                r = g * L + r16
                pu = (ivu[r16] % 2) * D
                pi = (ivi[r16] % 2) * D
                acc = (ubuf_v[r, pl.ds(pu, L)]
                       * ibuf_v[r, pl.ds(pi, L)])
                for k in range(1, D // L):
                    acc = acc + (ubuf_v[r, pl.ds(pu + k * L, L)]
                                 * ibuf_v[r, pl.ds(pi + k * L, L)])
                scr_v[pl.ds(r16 * L, L)] = acc
            x = plsc.load_gather(scr_v, [lanes * L])
            for k in range(1, L):
                x = x + plsc.load_gather(scr_v, [lanes * L + k])
            e = jnp.exp(-jnp.abs(x))
            out_v[pl.ds(rbase, L)] = jnp.where(
                x >= 0, 1.0 / (1.0 + e), e / (1.0 + e))
            return 0

        lax.fori_loop(0, GPC, group_body, 0)

    pltpu.sync_copy(out_v, out_hbm.at[pl.ds(base, BPW)])


def kernel(user, item, user_table, item_table):
    mesh = plsc.VectorSubcoreMesh(core_axis_name="c", subcore_axis_name="s")
    k = functools.partial(
        pl.kernel,
        mesh=mesh,
        compiler_params=pltpu.CompilerParams(
            needs_layout_passes=False, use_tc_tiling_on_sc=True),
        out_type=jax.ShapeDtypeStruct((B,), jnp.float32),
        scratch_types=[
            pltpu.VMEM((BPW,), jnp.int32),
            pltpu.VMEM((BPW,), jnp.int32),
            pltpu.VMEM((BPW,), jnp.int32),
            pltpu.VMEM((BPW,), jnp.int32),
            pltpu.VMEM((CH, 2 * D), jnp.float32),
            pltpu.VMEM((CH, 2 * D), jnp.float32),
            pltpu.VMEM((BPW,), jnp.float32),
            pltpu.VMEM((L * L,), jnp.float32),
            pltpu.SemaphoreType.DMA,
            pltpu.SemaphoreType.DMA,
        ],
    )(_sc_body)
    return k(user, item,
             user_table.reshape(500000, 2 * D), item_table.reshape(500000, 2 * D))
